# baseline (speedup 1.0000x reference)
import functools

import jax
import jax.numpy as jnp
from jax import lax
from jax.experimental import pallas as pl
from jax.experimental.pallas import tpu as pltpu
from jax.experimental.pallas import tpu_sc as plsc

N = 16384
W = 256
LANES = 16
NC = 2
NS = 16
NW = NC * NS

N_SC = 1024
N_TC = N - N_SC
RPW = N_SC // NW
CHUNK = min(128, RPW)
NCHUNK = RPW // CHUNK
WVEC = W // LANES

R_TC = 1024
assert N_TC % R_TC == 0 and RPW % 16 == 0 and RPW % CHUNK == 0


_GATHER_DNUMS = lax.GatherDimensionNumbers(
    offset_dims=(), collapsed_slice_dims=(0,), start_index_map=(0,))


def _lane_shuffle(x, idx):
    return lax.gather(x, idx[:, None], _GATHER_DNUMS, (1,),
                      mode=lax.GatherScatterMode.PROMISE_IN_BOUNDS)


def _combine(a, b, sh, lane_iota):
    fa = a + _lane_shuffle(a, lane_iota ^ sh)
    fb = b + _lane_shuffle(b, lane_iota ^ sh)
    return jnp.where((lane_iota & sh) == 0, fa, fb)


def _reduce_batch8(vs, lane_iota):
    for sh in (8, 4, 2):
        vs = [_combine(vs[2 * k], vs[2 * k + 1], sh, lane_iota)
              for k in range(len(vs) // 2)]
    return vs[0]


def _bitrev4(lane_iota):
    return (((lane_iota & 1) << 3) | ((lane_iota & 2) << 1)
            | ((lane_iota & 4) >> 1) | ((lane_iota & 8) >> 3))


def _finish16(za, zb, lane_iota, bitrev):
    z = _combine(za, zb, 1, lane_iota)
    return _lane_shuffle(z, bitrev)


def _sc_phase1(mem_hbm, key_hbm, dot_hbm, sq_hbm,
               key_v, buf0, buf1, dot_v, sq_v, sem0, sem1):
    wid = lax.axis_index("s") * NC + lax.axis_index("c")
    base = wid * RPW

    pltpu.sync_copy(key_hbm, key_v)
    kv = [key_v[pl.ds(LANES * j, LANES)] for j in range(WVEC)]

    bufs = (buf0, buf1)
    sems = (sem0, sem1)
    copies = [None, None]
    copies[0] = pltpu.async_copy(mem_hbm.at[pl.ds(base, CHUNK)], buf0, sem0)

    for c in range(NCHUNK):
        cur = c % 2
        if c + 1 < NCHUNK:
            copies[1 - cur] = pltpu.async_copy(
                mem_hbm.at[pl.ds(base + (c + 1) * CHUNK, CHUNK)],
                bufs[1 - cur], sems[1 - cur])
        copies[cur].wait()
        buf = bufs[cur]

        def group_body(g, _, buf=buf, off=c * CHUNK):
            lane_iota = lax.iota(jnp.int32, LANES)
            bitrev = _bitrev4(lane_iota)
            zd, zs = [], []
            for batch in range(2):
                daccs, saccs = [], []
                for i in range(8):
                    r = g * LANES + batch * 8 + i
                    v = buf[r, pl.ds(0, LANES)]
                    dacc = v * kv[0]
                    sacc = v * v
                    for j in range(1, WVEC):
                        v = buf[r, pl.ds(LANES * j, LANES)]
                        dacc = dacc + v * kv[j]
                        sacc = sacc + v * v
                    daccs.append(dacc)
                    saccs.append(sacc)
                zd.append(_reduce_batch8(daccs, lane_iota))
                zs.append(_reduce_batch8(saccs, lane_iota))
            dot_v[pl.ds(off + g * LANES, LANES)] = _finish16(
                zd[0], zd[1], lane_iota, bitrev)
            sq_v[pl.ds(off + g * LANES, LANES)] = _finish16(
                zs[0], zs[1], lane_iota, bitrev)
            return 0

        lax.fori_loop(0, CHUNK // LANES, group_body, 0)

    pltpu.sync_copy(dot_v, dot_hbm.at[pl.ds(base, RPW)])
    pltpu.sync_copy(sq_v, sq_hbm.at[pl.ds(base, RPW)])


_phase1_sc = functools.partial(
    pl.kernel,
    out_type=(jax.ShapeDtypeStruct((N_SC,), jnp.float32),
              jax.ShapeDtypeStruct((N_SC,), jnp.float32)),
    mesh=plsc.VectorSubcoreMesh(core_axis_name="c", subcore_axis_name="s"),
    scratch_types=(
        pltpu.VMEM((W,), jnp.float32),
        pltpu.VMEM((CHUNK, W), jnp.float32),
        pltpu.VMEM((CHUNK, W), jnp.float32),
        pltpu.VMEM((RPW,), jnp.float32),
        pltpu.VMEM((RPW,), jnp.float32),
        pltpu.SemaphoreType.DMA,
        pltpu.SemaphoreType.DMA,
    ),
)(_sc_phase1)


def _tc_phase1b(mem_ref, key_ref, dot_ref, sq_ref):
    mem = mem_ref[...]
    key = key_ref[...]
    dot_ref[0, 0, :] = jnp.sum(mem * key, axis=1)
    sq_ref[0, 0, :] = jnp.sum(mem * mem, axis=1)


def _tc_phase2(dot_ref, sq_ref, key_ref, strength_ref, agate_ref, wgate_ref,
               alloc_ref, out_ref):
    key = key_ref[...]
    key_norm = jnp.sqrt(jnp.sum(key * key))
    dots = dot_ref[...]
    mem_norm = jnp.sqrt(sq_ref[...])
    denom = jnp.maximum(mem_norm * key_norm, 1e-8)
    s = dots / denom * strength_ref[0, 0]
    m = jnp.max(s)
    e = jnp.exp(s - m)
    cw = e / jnp.sum(e)
    ag = agate_ref[0, 0]
    wg = wgate_ref[0, 0]
    out_ref[...] = wg * (ag * alloc_ref[...] + (1.0 - ag) * cw)


def kernel(memory, write_key, write_strength, allocation_gate, write_gate,
           allocation_weighting):
    dots_sc, sumsq_sc = _phase1_sc(memory[:N_SC], write_key)

    nblk = N_TC // R_TC
    dots_tc, sumsq_tc = pl.pallas_call(
        _tc_phase1b,
        grid=(nblk,),
        in_specs=[
            pl.BlockSpec((R_TC, W), lambda i: (i, 0)),
            pl.BlockSpec((1, W), lambda i: (0, 0)),
        ],
        out_specs=[
            pl.BlockSpec((1, 1, R_TC), lambda i: (i, 0, 0)),
            pl.BlockSpec((1, 1, R_TC), lambda i: (i, 0, 0)),
        ],
        out_shape=(
            jax.ShapeDtypeStruct((nblk, 1, R_TC), jnp.float32),
            jax.ShapeDtypeStruct((nblk, 1, R_TC), jnp.float32),
        ),
    )(memory[N_SC:], write_key.reshape(1, W))

    dots = jnp.concatenate([dots_sc, dots_tc.reshape(N_TC)])
    sumsq = jnp.concatenate([sumsq_sc, sumsq_tc.reshape(N_TC)])

    out2d = pl.pallas_call(
        _tc_phase2,
        out_shape=jax.ShapeDtypeStruct((128, 128), jnp.float32),
    )(
        dots.reshape(128, 128),
        sumsq.reshape(128, 128),
        write_key.reshape(2, 128),
        write_strength.reshape(1, 1),
        allocation_gate.reshape(1, 1),
        write_gate.reshape(1, 1),
        allocation_weighting.reshape(128, 128),
    )
    return out2d.reshape(N)


# speedup vs baseline: 1.4876x; 1.4876x over previous
import functools

import jax
import jax.numpy as jnp
from jax import lax
from jax.experimental import pallas as pl
from jax.experimental.pallas import tpu as pltpu
from jax.experimental.pallas import tpu_sc as plsc

N = 16384
W = 256
LANES = 16
NC = 2
NS = 16
NW = NC * NS

N_SC = 1024
N_TC = N - N_SC
RPW = N_SC // NW
CHUNK = min(128, RPW)
NCHUNK = RPW // CHUNK
WVEC = W // LANES

R_TC = 1024
assert N_TC % R_TC == 0 and RPW % 16 == 0 and RPW % CHUNK == 0


_GATHER_DNUMS = lax.GatherDimensionNumbers(
    offset_dims=(), collapsed_slice_dims=(0,), start_index_map=(0,))


def _lane_shuffle(x, idx):
    return lax.gather(x, idx[:, None], _GATHER_DNUMS, (1,),
                      mode=lax.GatherScatterMode.PROMISE_IN_BOUNDS)


def _combine(a, b, sh, lane_iota):
    fa = a + _lane_shuffle(a, lane_iota ^ sh)
    fb = b + _lane_shuffle(b, lane_iota ^ sh)
    return jnp.where((lane_iota & sh) == 0, fa, fb)


def _reduce_batch8(vs, lane_iota):
    for sh in (8, 4, 2):
        vs = [_combine(vs[2 * k], vs[2 * k + 1], sh, lane_iota)
              for k in range(len(vs) // 2)]
    return vs[0]


def _bitrev4(lane_iota):
    return (((lane_iota & 1) << 3) | ((lane_iota & 2) << 1)
            | ((lane_iota & 4) >> 1) | ((lane_iota & 8) >> 3))


def _finish16(za, zb, lane_iota, bitrev):
    z = _combine(za, zb, 1, lane_iota)
    return _lane_shuffle(z, bitrev)


def _sc_phase1(mem_hbm, key_hbm, dot_hbm, sq_hbm,
               key_v, buf0, buf1, dot_v, sq_v, sem0, sem1):
    wid = lax.axis_index("s") * NC + lax.axis_index("c")
    base = wid * RPW

    bufs = (buf0, buf1)
    sems = (sem0, sem1)
    copies = [None, None]
    copies[0] = pltpu.async_copy(mem_hbm.at[pl.ds(base, CHUNK)], buf0, sem0)
    pltpu.sync_copy(key_hbm, key_v)
    kv = [key_v[pl.ds(LANES * j, LANES)] for j in range(WVEC)]

    for c in range(NCHUNK):
        cur = c % 2
        if c + 1 < NCHUNK:
            copies[1 - cur] = pltpu.async_copy(
                mem_hbm.at[pl.ds(base + (c + 1) * CHUNK, CHUNK)],
                bufs[1 - cur], sems[1 - cur])
        copies[cur].wait()
        buf = bufs[cur]

        def group_body(g, _, buf=buf, off=c * CHUNK):
            lane_iota = lax.iota(jnp.int32, LANES)
            bitrev = _bitrev4(lane_iota)
            zd, zs = [], []
            for batch in range(2):
                daccs, saccs = [], []
                for i in range(8):
                    r = g * LANES + batch * 8 + i
                    v = buf[r, pl.ds(0, LANES)]
                    dacc = v * kv[0]
                    sacc = v * v
                    for j in range(1, WVEC):
                        v = buf[r, pl.ds(LANES * j, LANES)]
                        dacc = dacc + v * kv[j]
                        sacc = sacc + v * v
                    daccs.append(dacc)
                    saccs.append(sacc)
                zd.append(_reduce_batch8(daccs, lane_iota))
                zs.append(_reduce_batch8(saccs, lane_iota))
            dot_v[pl.ds(off + g * LANES, LANES)] = _finish16(
                zd[0], zd[1], lane_iota, bitrev)
            sq_v[pl.ds(off + g * LANES, LANES)] = _finish16(
                zs[0], zs[1], lane_iota, bitrev)
            return 0

        lax.fori_loop(0, CHUNK // LANES, group_body, 0)

    o1 = pltpu.async_copy(dot_v, dot_hbm.at[pl.ds(base, RPW)], sem0)
    o2 = pltpu.async_copy(sq_v, sq_hbm.at[pl.ds(base, RPW)], sem1)
    o1.wait()
    o2.wait()


_phase1_sc = functools.partial(
    pl.kernel,
    out_type=(jax.ShapeDtypeStruct((N_SC,), jnp.float32),
              jax.ShapeDtypeStruct((N_SC,), jnp.float32)),
    mesh=plsc.VectorSubcoreMesh(core_axis_name="c", subcore_axis_name="s"),
    scratch_types=(
        pltpu.VMEM((W,), jnp.float32),
        pltpu.VMEM((CHUNK, W), jnp.float32),
        pltpu.VMEM((CHUNK, W), jnp.float32),
        pltpu.VMEM((RPW,), jnp.float32),
        pltpu.VMEM((RPW,), jnp.float32),
        pltpu.SemaphoreType.DMA,
        pltpu.SemaphoreType.DMA,
    ),
)(_sc_phase1)


def _tc_phase1b(mem_ref, key_ref, dot_ref, sq_ref):
    mem = mem_ref[...]
    key8 = key_ref[...]
    ones8 = jnp.ones((W, 8), jnp.float32)
    dots = lax.dot_general(mem, key8, (((1,), (0,)), ((), ())),
                           preferred_element_type=jnp.float32)
    sq = lax.dot_general(mem * mem, ones8, (((1,), (0,)), ((), ())),
                         preferred_element_type=jnp.float32)
    dot_ref[0, 0, :] = dots.T[0]
    sq_ref[0, 0, :] = sq.T[0]


def _tc_phase2(dsc_ref, ssc_ref, dtc_ref, stc_ref, key_ref, strength_ref,
               agate_ref, wgate_ref, alloc_ref, out_ref):
    key = key_ref[...]
    key_norm = jnp.sqrt(jnp.sum(key * key))
    strength = strength_ref[0, 0]
    ag = agate_ref[0, 0]
    wg = wgate_ref[0, 0]

    def weighted(dots, sumsq):
        denom = jnp.maximum(jnp.sqrt(sumsq) * key_norm, 1e-8)
        return dots / denom * strength

    s_sc = weighted(dsc_ref[...], ssc_ref[...])
    s_tc = weighted(dtc_ref[...], stc_ref[...])
    m = jnp.maximum(jnp.max(s_sc), jnp.max(s_tc))
    e_sc = jnp.exp(s_sc - m)
    e_tc = jnp.exp(s_tc - m)
    z = jnp.sum(e_sc) + jnp.sum(e_tc)
    scale_a = wg * ag
    scale_c = wg * (1.0 - ag) / z
    nsc_rows = N_SC // 128
    alloc = alloc_ref[...]
    out_ref[:nsc_rows, :] = scale_a * alloc[:nsc_rows, :] + scale_c * e_sc
    out_ref[nsc_rows:, :] = scale_a * alloc[nsc_rows:, :] + scale_c * e_tc


def kernel(memory, write_key, write_strength, allocation_gate, write_gate,
           allocation_weighting):
    dots_sc, sumsq_sc = _phase1_sc(memory, write_key)

    nblk = N_TC // R_TC
    blk0 = N_SC // R_TC
    key8 = jnp.broadcast_to(write_key[:, None], (W, 8))
    dots_tc, sumsq_tc = pl.pallas_call(
        _tc_phase1b,
        grid=(nblk,),
        in_specs=[
            pl.BlockSpec((R_TC, W), lambda i: (i + blk0, 0)),
            pl.BlockSpec((W, 8), lambda i: (0, 0)),
        ],
        out_specs=[
            pl.BlockSpec((1, 1, R_TC), lambda i: (i, 0, 0)),
            pl.BlockSpec((1, 1, R_TC), lambda i: (i, 0, 0)),
        ],
        out_shape=(
            jax.ShapeDtypeStruct((nblk, 1, R_TC), jnp.float32),
            jax.ShapeDtypeStruct((nblk, 1, R_TC), jnp.float32),
        ),
    )(memory, key8)

    out2d = pl.pallas_call(
        _tc_phase2,
        out_shape=jax.ShapeDtypeStruct((128, 128), jnp.float32),
    )(
        dots_sc.reshape(N_SC // 128, 128),
        sumsq_sc.reshape(N_SC // 128, 128),
        dots_tc.reshape(N_TC // 128, 128),
        sumsq_tc.reshape(N_TC // 128, 128),
        write_key.reshape(2, 128),
        write_strength.reshape(1, 1),
        allocation_gate.reshape(1, 1),
        write_gate.reshape(1, 1),
        allocation_weighting.reshape(128, 128),
    )
    return out2d.reshape(N)


# speedup vs baseline: 2.0219x; 1.3592x over previous
import functools

import jax
import jax.numpy as jnp
from jax import lax
from jax.experimental import pallas as pl
from jax.experimental.pallas import tpu as pltpu
from jax.experimental.pallas import tpu_sc as plsc

N = 16384
W = 256
LANES = 16
NC = 2
NS = 16
NW = NC * NS

N_SC = 1024
N_TC = N - N_SC
RPW = N_SC // NW
CHUNK = min(128, RPW)
NCHUNK = RPW // CHUNK
WVEC = W // LANES

R_TC = 1024
assert N_TC % R_TC == 0 and RPW % 16 == 0 and RPW % CHUNK == 0


_GATHER_DNUMS = lax.GatherDimensionNumbers(
    offset_dims=(), collapsed_slice_dims=(0,), start_index_map=(0,))


def _lane_shuffle(x, idx):
    return lax.gather(x, idx[:, None], _GATHER_DNUMS, (1,),
                      mode=lax.GatherScatterMode.PROMISE_IN_BOUNDS)


def _combine(a, b, sh, lane_iota):
    fa = a + _lane_shuffle(a, lane_iota ^ sh)
    fb = b + _lane_shuffle(b, lane_iota ^ sh)
    return jnp.where((lane_iota & sh) == 0, fa, fb)


def _reduce_batch8(vs, lane_iota):
    for sh in (8, 4, 2):
        vs = [_combine(vs[2 * k], vs[2 * k + 1], sh, lane_iota)
              for k in range(len(vs) // 2)]
    return vs[0]


def _bitrev4(lane_iota):
    return (((lane_iota & 1) << 3) | ((lane_iota & 2) << 1)
            | ((lane_iota & 4) >> 1) | ((lane_iota & 8) >> 3))


def _finish16(za, zb, lane_iota, bitrev):
    z = _combine(za, zb, 1, lane_iota)
    return _lane_shuffle(z, bitrev)


def _sc_phase1(mem_hbm, key_hbm, dot_hbm, sq_hbm,
               key_v, buf0, buf1, dot_v, sq_v, sem0, sem1):
    wid = lax.axis_index("s") * NC + lax.axis_index("c")
    base = wid * RPW

    bufs = (buf0, buf1)
    sems = (sem0, sem1)
    copies = [None, None]
    copies[0] = pltpu.async_copy(mem_hbm.at[pl.ds(base, CHUNK)], buf0, sem0)
    pltpu.sync_copy(key_hbm, key_v)
    kv = [key_v[pl.ds(LANES * j, LANES)] for j in range(WVEC)]

    for c in range(NCHUNK):
        cur = c % 2
        if c + 1 < NCHUNK:
            copies[1 - cur] = pltpu.async_copy(
                mem_hbm.at[pl.ds(base + (c + 1) * CHUNK, CHUNK)],
                bufs[1 - cur], sems[1 - cur])
        copies[cur].wait()
        buf = bufs[cur]

        def group_body(g, _, buf=buf, off=c * CHUNK):
            lane_iota = lax.iota(jnp.int32, LANES)
            bitrev = _bitrev4(lane_iota)
            zd, zs = [], []
            for batch in range(2):
                daccs, saccs = [], []
                for i in range(8):
                    r = g * LANES + batch * 8 + i
                    v = buf[r, pl.ds(0, LANES)]
                    dacc = v * kv[0]
                    sacc = v * v
                    for j in range(1, WVEC):
                        v = buf[r, pl.ds(LANES * j, LANES)]
                        dacc = dacc + v * kv[j]
                        sacc = sacc + v * v
                    daccs.append(dacc)
                    saccs.append(sacc)
                zd.append(_reduce_batch8(daccs, lane_iota))
                zs.append(_reduce_batch8(saccs, lane_iota))
            dot_v[pl.ds(off + g * LANES, LANES)] = _finish16(
                zd[0], zd[1], lane_iota, bitrev)
            sq_v[pl.ds(off + g * LANES, LANES)] = _finish16(
                zs[0], zs[1], lane_iota, bitrev)
            return 0

        lax.fori_loop(0, CHUNK // LANES, group_body, 0)

    o1 = pltpu.async_copy(dot_v, dot_hbm.at[pl.ds(base, RPW)], sem0)
    o2 = pltpu.async_copy(sq_v, sq_hbm.at[pl.ds(base, RPW)], sem1)
    o1.wait()
    o2.wait()


_phase1_sc = functools.partial(
    pl.kernel,
    out_type=(jax.ShapeDtypeStruct((N_SC,), jnp.float32),
              jax.ShapeDtypeStruct((N_SC,), jnp.float32)),
    mesh=plsc.VectorSubcoreMesh(core_axis_name="c", subcore_axis_name="s"),
    scratch_types=(
        pltpu.VMEM((W,), jnp.float32),
        pltpu.VMEM((CHUNK, W), jnp.float32),
        pltpu.VMEM((CHUNK, W), jnp.float32),
        pltpu.VMEM((RPW,), jnp.float32),
        pltpu.VMEM((RPW,), jnp.float32),
        pltpu.SemaphoreType.DMA,
        pltpu.SemaphoreType.DMA,
    ),
)(_sc_phase1)


def _tc_phase1b(mem_ref, key_ref, dot_ref, sq_ref):
    mem = mem_ref[...]
    key8 = key_ref[...]
    ones8 = jnp.ones((W, 8), jnp.float32)
    dots = lax.dot_general(mem, key8, (((1,), (0,)), ((), ())),
                           preferred_element_type=jnp.float32)
    sq = lax.dot_general(mem * mem, ones8, (((1,), (0,)), ((), ())),
                         preferred_element_type=jnp.float32)
    dot_ref[0, 0, :] = dots.T[0]
    sq_ref[0, 0, :] = sq.T[0]


def _tc_phase2(dsc_ref, ssc_ref, dtc_ref, stc_ref, key_ref, strength_ref,
               agate_ref, wgate_ref, alloc_ref, out_ref):
    key = key_ref[...]
    key_norm = jnp.sqrt(jnp.sum(key * key))
    strength = strength_ref[0, 0]
    ag = agate_ref[0, 0]
    wg = wgate_ref[0, 0]

    def weighted(dots, sumsq):
        denom = jnp.maximum(jnp.sqrt(sumsq) * key_norm, 1e-8)
        return dots / denom * strength

    s_sc = weighted(dsc_ref[...], ssc_ref[...])
    s_tc = weighted(dtc_ref[...], stc_ref[...])
    m = jnp.maximum(jnp.max(s_sc), jnp.max(s_tc))
    e_sc = jnp.exp(s_sc - m)
    e_tc = jnp.exp(s_tc - m)
    z = jnp.sum(e_sc) + jnp.sum(e_tc)
    scale_a = wg * ag
    scale_c = wg * (1.0 - ag) / z
    nsc_rows = N_SC // 128
    alloc = alloc_ref[...]
    out_ref[:nsc_rows, :] = scale_a * alloc[:nsc_rows, :] + scale_c * e_sc
    out_ref[nsc_rows:, :] = scale_a * alloc[nsc_rows:, :] + scale_c * e_tc


_USE_SC = False


def kernel(memory, write_key, write_strength, allocation_gate, write_gate,
           allocation_weighting):
    if _USE_SC:
        dots_sc, sumsq_sc = _phase1_sc(memory, write_key)
        nblk, blk0 = N_TC // R_TC, N_SC // R_TC
    else:
        nblk, blk0 = N // R_TC, 0
    key8 = jnp.broadcast_to(write_key[:, None], (W, 8))
    dots_tc, sumsq_tc = pl.pallas_call(
        _tc_phase1b,
        grid=(nblk,),
        in_specs=[
            pl.BlockSpec((R_TC, W), lambda i: (i + blk0, 0)),
            pl.BlockSpec((W, 8), lambda i: (0, 0)),
        ],
        out_specs=[
            pl.BlockSpec((1, 1, R_TC), lambda i: (i, 0, 0)),
            pl.BlockSpec((1, 1, R_TC), lambda i: (i, 0, 0)),
        ],
        out_shape=(
            jax.ShapeDtypeStruct((nblk, 1, R_TC), jnp.float32),
            jax.ShapeDtypeStruct((nblk, 1, R_TC), jnp.float32),
        ),
    )(memory, key8)

    if _USE_SC:
        dsc = dots_sc.reshape(N_SC // 128, 128)
        ssc = sumsq_sc.reshape(N_SC // 128, 128)
        dtc = dots_tc.reshape(N_TC // 128, 128)
        stc = sumsq_tc.reshape(N_TC // 128, 128)
    else:
        d_all = dots_tc.reshape(N // 128, 128)
        s_all = sumsq_tc.reshape(N // 128, 128)
        dsc, dtc = d_all[:N_SC // 128], d_all[N_SC // 128:]
        ssc, stc = s_all[:N_SC // 128], s_all[N_SC // 128:]

    out2d = pl.pallas_call(
        _tc_phase2,
        out_shape=jax.ShapeDtypeStruct((128, 128), jnp.float32),
    )(
        dsc, ssc, dtc, stc,
        write_key.reshape(2, 128),
        write_strength.reshape(1, 1),
        allocation_gate.reshape(1, 1),
        write_gate.reshape(1, 1),
        allocation_weighting.reshape(128, 128),
    )
    return out2d.reshape(N)
